# baseline (speedup 1.0000x reference)
import functools

import jax
import jax.numpy as jnp
from jax import lax
from jax.experimental import pallas as pl
from jax.experimental.pallas import tpu as pltpu
from jax.experimental.pallas import tpu_sc as plsc

B, A = 4096, 64
NC, NS, L = 2, 16, 16
NW = NC * NS
TOTAL = B * A * 2
PER_W = TOTAL // NW
UNROLL = 8
STEPS = PER_W // (L * UNROLL)

_mesh = plsc.VectorSubcoreMesh(core_axis_name="c", subcore_axis_name="s")

_GATHER_DNUMS = lax.GatherDimensionNumbers(
    offset_dims=(), collapsed_slice_dims=(0,), start_index_map=(0,)
)


def _vgather(src, idx):
    return lax.gather(
        src, idx[:, None], _GATHER_DNUMS, slice_sizes=(1,),
        mode=lax.GatherScatterMode.PROMISE_IN_BOUNDS,
    )


@functools.partial(
    pl.kernel,
    mesh=_mesh,
    out_type=jax.ShapeDtypeStruct((TOTAL,), jnp.int32),
    scratch_types=[
        pltpu.VMEM((PER_W,), jnp.int32),
        pltpu.VMEM((L,), jnp.int32),
    ],
)
def _convert_sc(words_hbm, lut_hbm, out_hbm, buf, lutbuf):
    wid = lax.axis_index("s") * NC + lax.axis_index("c")
    base = wid * PER_W
    pltpu.sync_copy(lut_hbm, lutbuf)
    pltpu.sync_copy(words_hbm.at[pl.ds(base, PER_W)], buf)
    lut = lutbuf[...]
    iota = lax.iota(jnp.int32, L)
    idx_dup = iota & jnp.int32(-2)
    odd = (iota & jnp.int32(1)) == jnp.int32(1)

    def step(i, carry):
        for u in range(UNROLL):
            off = (i * jnp.int32(UNROLL) + jnp.int32(u)) * jnp.int32(L)
            v = buf[pl.ds(off, L)]
            lows = _vgather(v, idx_dup)
            clipped = jnp.minimum(jnp.maximum(lows, jnp.int32(0)), jnp.int32(8))
            conv = _vgather(lut, clipped)
            res = jnp.where(odd, lax.shift_right_arithmetic(conv, jnp.int32(31)), conv)
            buf[pl.ds(off, L)] = res
        return carry

    lax.fori_loop(jnp.int32(0), jnp.int32(STEPS), step, jnp.int32(0))
    pltpu.sync_copy(buf, out_hbm.at[pl.ds(base, PER_W)])


def kernel(species, coordinates, conv_table):
    words = lax.bitcast_convert_type(species, jnp.int32).reshape(TOTAL)
    lut = jnp.concatenate(
        [conv_table.astype(jnp.int32), jnp.full((L - 9,), -1, jnp.int32)]
    )
    out32 = _convert_sc(words, lut)
    species_idx = lax.bitcast_convert_type(out32.reshape(B, A, 2), jnp.int64)
    return species_idx, coordinates


# speedup vs baseline: 1.0029x; 1.0029x over previous
import functools

import jax
import jax.numpy as jnp
from jax import lax
from jax.experimental import pallas as pl
from jax.experimental.pallas import tpu as pltpu
from jax.experimental.pallas import tpu_sc as plsc

B, A = 4096, 64
NC, NS, L = 2, 16, 16
NW = NC * NS
TOTAL = B * A * 2
PER_W = TOTAL // NW
UNROLL = 8
STEPS = PER_W // (L * UNROLL)

_mesh = plsc.VectorSubcoreMesh(core_axis_name="c", subcore_axis_name="s")

_GATHER_DNUMS = lax.GatherDimensionNumbers(
    offset_dims=(), collapsed_slice_dims=(0,), start_index_map=(0,)
)


def _vgather(src, idx):
    return lax.gather(
        src, idx[:, None], _GATHER_DNUMS, slice_sizes=(1,),
        mode=lax.GatherScatterMode.PROMISE_IN_BOUNDS,
    )


@functools.partial(
    pl.kernel,
    mesh=_mesh,
    out_type=jax.ShapeDtypeStruct((TOTAL,), jnp.int32),
    scratch_types=[
        pltpu.VMEM((PER_W,), jnp.int32),
        pltpu.VMEM((PER_W,), jnp.int32),
        pltpu.VMEM((L,), jnp.int32),
    ],
)
def _convert_sc(words_hbm, lut_hbm, out_hbm, inbuf, outbuf, lutbuf):
    wid = lax.axis_index("s") * NC + lax.axis_index("c")
    base = wid * PER_W
    pltpu.sync_copy(lut_hbm, lutbuf)
    pltpu.sync_copy(words_hbm.at[pl.ds(base, PER_W)], inbuf)
    lut = lutbuf[...]
    iota = lax.iota(jnp.int32, L)
    idx_dup = iota & jnp.int32(-2)
    odd = (iota & jnp.int32(1)) == jnp.int32(1)

    @plsc.parallel_loop(jnp.int32(0), jnp.int32(PER_W), step=jnp.int32(L), unroll=UNROLL)
    def _body(off):
        v = inbuf[pl.ds(off, L)]
        lows = _vgather(v, idx_dup)
        clipped = jnp.minimum(jnp.maximum(lows, jnp.int32(0)), jnp.int32(8))
        conv = _vgather(lut, clipped)
        res = jnp.where(odd, lax.shift_right_arithmetic(conv, jnp.int32(31)), conv)
        outbuf[pl.ds(off, L)] = res

    pltpu.sync_copy(outbuf, out_hbm.at[pl.ds(base, PER_W)])


def kernel(species, coordinates, conv_table):
    words = lax.bitcast_convert_type(species, jnp.int32).reshape(TOTAL)
    lut = jnp.concatenate(
        [conv_table.astype(jnp.int32), jnp.full((L - 9,), -1, jnp.int32)]
    )
    out32 = _convert_sc(words, lut)
    species_idx = lax.bitcast_convert_type(out32.reshape(B, A, 2), jnp.int64)
    return species_idx, coordinates


# speedup vs baseline: 10.3858x; 10.3557x over previous
import functools

import jax
import jax.numpy as jnp
from jax import lax
from jax.experimental import pallas as pl
from jax.experimental.pallas import tpu as pltpu
from jax.experimental.pallas import tpu_sc as plsc

B, A = 4096, 64
NC, NS, L = 2, 16, 16
NW = NC * NS
TOTAL = B * A
PER_W = TOTAL // NW
UNROLL = 8

_mesh = plsc.VectorSubcoreMesh(core_axis_name="c", subcore_axis_name="s")

_GATHER_DNUMS = lax.GatherDimensionNumbers(
    offset_dims=(), collapsed_slice_dims=(0,), start_index_map=(0,)
)


def _vgather(src, idx):
    return lax.gather(
        src, idx[:, None], _GATHER_DNUMS, slice_sizes=(1,),
        mode=lax.GatherScatterMode.PROMISE_IN_BOUNDS,
    )


@functools.partial(
    pl.kernel,
    mesh=_mesh,
    out_type=jax.ShapeDtypeStruct((TOTAL,), jnp.int32),
    scratch_types=[
        pltpu.VMEM((PER_W,), jnp.int32),
        pltpu.VMEM((PER_W,), jnp.int32),
        pltpu.VMEM((L,), jnp.int32),
    ],
)
def _convert_sc(species_hbm, lut_hbm, out_hbm, inbuf, outbuf, lutbuf):
    wid = lax.axis_index("s") * jnp.int32(NC) + lax.axis_index("c")
    base = wid * jnp.int32(PER_W)
    pltpu.sync_copy(lut_hbm, lutbuf)
    pltpu.sync_copy(species_hbm.at[pl.ds(base, PER_W)], inbuf)
    lut = lutbuf[...]

    @plsc.parallel_loop(jnp.int32(0), jnp.int32(PER_W), step=jnp.int32(L),
                        unroll=UNROLL)
    def _body(off):
        v = inbuf[pl.ds(off, L)]
        clipped = jnp.minimum(jnp.maximum(v, jnp.int32(0)), jnp.int32(8))
        outbuf[pl.ds(off, L)] = _vgather(lut, clipped)

    pltpu.sync_copy(outbuf, out_hbm.at[pl.ds(base, PER_W)])


def kernel(species, coordinates, conv_table):
    species32 = species.astype(jnp.int32).reshape(TOTAL)
    lut = jnp.concatenate(
        [conv_table.astype(jnp.int32), jnp.full((L - 9,), -1, jnp.int32)]
    )
    out32 = _convert_sc(species32, lut)
    species_idx = out32.reshape(B, A).astype(jnp.int64)
    return species_idx, coordinates


# speedup vs baseline: 14.0419x; 1.3520x over previous
import functools

import jax
import jax.numpy as jnp
from jax import lax
from jax.experimental import pallas as pl
from jax.experimental.pallas import tpu as pltpu
from jax.experimental.pallas import tpu_sc as plsc

B, A = 4096, 64
NC, NS, L = 2, 16, 16
NW = NC * NS
TOTAL = B * A
PER_W = TOTAL // NW
UNROLL = 8

_mesh = plsc.VectorSubcoreMesh(core_axis_name="c", subcore_axis_name="s")

_GATHER_DNUMS = lax.GatherDimensionNumbers(
    offset_dims=(), collapsed_slice_dims=(0,), start_index_map=(0,)
)


def _vgather(src, idx):
    return lax.gather(
        src, idx[:, None], _GATHER_DNUMS, slice_sizes=(1,),
        mode=lax.GatherScatterMode.PROMISE_IN_BOUNDS,
    )


ROWS_W = A // NW


@functools.partial(
    pl.kernel,
    mesh=_mesh,
    out_type=jax.ShapeDtypeStruct((A, B), jnp.int32),
    scratch_types=[
        pltpu.VMEM((ROWS_W, B), jnp.int32),
        pltpu.VMEM((ROWS_W, B), jnp.int32),
        pltpu.VMEM((L,), jnp.int32),
    ],
)
def _convert_sc(species_hbm, lut_hbm, out_hbm, inbuf, outbuf, lutbuf):
    wid = lax.axis_index("s") * jnp.int32(NC) + lax.axis_index("c")
    base = wid * jnp.int32(ROWS_W)
    pltpu.sync_copy(lut_hbm, lutbuf)
    pltpu.sync_copy(species_hbm.at[pl.ds(base, ROWS_W)], inbuf)
    lut = lutbuf[...]

    for r in range(ROWS_W):
        @plsc.parallel_loop(jnp.int32(0), jnp.int32(B), step=jnp.int32(L),
                            unroll=UNROLL)
        def _body(off, _r=r):
            v = inbuf[_r, pl.ds(off, L)]
            clipped = jnp.minimum(jnp.maximum(v, jnp.int32(0)), jnp.int32(8))
            outbuf[_r, pl.ds(off, L)] = _vgather(lut, clipped)

    pltpu.sync_copy(outbuf, out_hbm.at[pl.ds(base, ROWS_W)])


def kernel(species, coordinates, conv_table):
    species32 = species.astype(jnp.int32).T
    lut = jnp.concatenate(
        [conv_table.astype(jnp.int32), jnp.full((L - 9,), -1, jnp.int32)]
    )
    out32 = _convert_sc(species32, lut)
    species_idx = out32.T.astype(jnp.int64)
    return species_idx, coordinates
